# Initial kernel scaffold; baseline (speedup 1.0000x reference)
import jax
import jax.numpy as jnp
from jax.experimental import pallas as pl


def kernel(original_pts, query_pts, W1, b1, W2, b2, W3, b3, Wg, bg, Wr1, br1, Wr2, br2, Wr3, br3):
    raise NotImplementedError("write your pallas kernel here")


# speedup vs baseline: 13.6283x; 13.6283x over previous
import jax
import jax.numpy as jnp
from jax import lax
from jax.experimental import pallas as pl
from jax.experimental.pallas import tpu as pltpu
from jax.experimental.pallas import tpu_sc as plsc

B, N, M = 2, 4096, 4096
K = 3
CF = 448
CP = 512
MB = 512
HI = lax.Precision.HIGHEST


def _bf(x):
    return x.astype(jnp.bfloat16).astype(jnp.float32)


def _features_body(ptsT_ref, W1_ref, b1_ref, W2_ref, b2_ref, W3_ref, b3_ref,
                   Wg_ref, bg_ref, Wr1g_ref, br1_ref,
                   fcat_ref, gterm_ref):
    pts = ptsT_ref[0]
    f1 = jnp.maximum(
        jnp.dot(pts, W1_ref[...].T, preferred_element_type=jnp.float32,
                precision=HI) + b1_ref[...][None, :], 0.0)
    f2 = jnp.maximum(
        jnp.dot(f1, W2_ref[...].T, preferred_element_type=jnp.float32,
                precision=HI) + b2_ref[...][None, :], 0.0)
    f3 = jnp.maximum(
        jnp.dot(f2, W3_ref[...].T, preferred_element_type=jnp.float32,
                precision=HI) + b3_ref[...][None, :], 0.0)
    fcat_ref[0] = jnp.concatenate(
        [f1, f2, f3, jnp.zeros((N, CP - CF), jnp.float32)], axis=1)
    g = jnp.max(f3, axis=0)
    g1024 = (jnp.dot(g[None, :], Wg_ref[...].T,
                     preferred_element_type=jnp.float32, precision=HI)
             + bg_ref[...][None, :])
    g1024r = g1024.astype(jnp.bfloat16).astype(jnp.float32)
    gterm_ref[0] = (jnp.dot(g1024r, Wr1g_ref[...].T,
                            preferred_element_type=jnp.float32, precision=HI)
                    + br1_ref[...][None, :])


def _features(ptsT, W1, b1, W2, b2, W3, b3, Wg, bg, Wr1g, br1):
    rep = lambda x: pl.BlockSpec(x.shape, lambda b: (0,) * x.ndim)
    return pl.pallas_call(
        _features_body,
        grid=(B,),
        in_specs=[
            pl.BlockSpec((1, N, 3), lambda b: (b, 0, 0)),
            rep(W1), rep(b1), rep(W2), rep(b2), rep(W3), rep(b3),
            rep(Wg), rep(bg), rep(Wr1g), rep(br1),
        ],
        out_specs=[
            pl.BlockSpec((1, N, CP), lambda b: (b, 0, 0)),
            pl.BlockSpec((1, 1, 256), lambda b: (b, 0, 0)),
        ],
        out_shape=[
            jax.ShapeDtypeStruct((B, N, CP), jnp.float32),
            jax.ShapeDtypeStruct((B, 1, 256), jnp.float32),
        ],
    )(ptsT, W1, b1, W2, b2, W3, b3, Wg, bg, Wr1g, br1)


def _knn_body(pts_ref, ptsT_ref, qT_ref, idx_ref, w_ref):
    b = pl.program_id(0)
    pts = pts_ref[0]
    ptsT = ptsT_ref[0]
    q = qT_ref[0]
    pn = jnp.sum(pts * pts, axis=0)
    qn = jnp.sum(q * q, axis=1)
    inner = jnp.dot(q.astype(jnp.bfloat16), pts.astype(jnp.bfloat16),
                    preferred_element_type=jnp.float32)
    d2 = qn[:, None] + pn[None, :] - 2.0 * inner
    iota = lax.broadcasted_iota(jnp.int32, (MB, N), 1)
    big = jnp.float32(jnp.inf)
    ids = []
    dist = []
    for _ in range(K):
        m = jnp.min(d2, axis=1, keepdims=True)
        hit = d2 <= m
        idxk = jnp.min(jnp.where(hit, iota, N), axis=1)
        ids.append(idxk)
        sel = iota == idxk[:, None]
        onehot = jnp.where(sel, 1.0, 0.0)
        coords = jnp.dot(onehot, ptsT, preferred_element_type=jnp.float32,
                         precision=HI)
        diff = coords - q
        dist.append(jnp.sqrt(jnp.sum(diff * diff, axis=1) + 1e-12))
        d2 = jnp.where(sel, big, d2)
    recip = [1.0 / (dd + 1e-08) for dd in dist]
    norm = recip[0] + recip[1] + recip[2]
    idx_ref[0] = jnp.stack(ids, axis=1) + b * N
    w_ref[0] = jnp.stack([r / norm for r in recip], axis=1)


def _knn(pts, ptsT, queryT):
    return pl.pallas_call(
        _knn_body,
        grid=(B, M // MB),
        in_specs=[
            pl.BlockSpec((1, 3, N), lambda b, m: (b, 0, 0)),
            pl.BlockSpec((1, N, 3), lambda b, m: (b, 0, 0)),
            pl.BlockSpec((1, MB, 3), lambda b, m: (b, m, 0)),
        ],
        out_specs=[
            pl.BlockSpec((1, MB, K), lambda b, m: (b, m, 0)),
            pl.BlockSpec((1, MB, K), lambda b, m: (b, m, 0)),
        ],
        out_shape=[
            jax.ShapeDtypeStruct((B, M, K), jnp.int32),
            jax.ShapeDtypeStruct((B, M, K), jnp.float32),
        ],
    )(pts, ptsT, queryT)


NW = 32
QPW = B * M // NW
QC = 32
NCHUNK = QPW // QC
RPC = QC * K


def _gather_body(table_hbm, idx_hbm, wexp_hbm, out_hbm,
                 idx_v, w_v, rows_v, out_v, gsem):
    cid = lax.axis_index("c")
    sid = lax.axis_index("s")
    wid = sid * 2 + cid
    qbase = wid * QPW

    def chunk(ci, carry):
        q0 = qbase + ci * QC
        pltpu.sync_copy(idx_hbm.at[pl.ds(q0 * K, RPC)], idx_v)
        pltpu.sync_copy(wexp_hbm.at[pl.ds(q0 * K, RPC)], w_v)
        pltpu.async_copy(table_hbm.at[idx_v], rows_v, gsem).wait()

        def query(qi, carry2):
            for d in range(CP // 16):
                sl = pl.ds(d * 16, 16)
                acc = w_v[qi * K, :] * rows_v[qi * K, sl]
                acc += w_v[qi * K + 1, :] * rows_v[qi * K + 1, sl]
                acc += w_v[qi * K + 2, :] * rows_v[qi * K + 2, sl]
                out_v[qi, sl] = acc
            return carry2

        lax.fori_loop(0, QC, query, 0)
        pltpu.sync_copy(out_v, out_hbm.at[pl.ds(q0, QC)])
        return carry

    lax.fori_loop(0, NCHUNK, chunk, 0)


def _sc_gather(table, idx_flat, wexp):
    mesh = plsc.VectorSubcoreMesh(core_axis_name="c", subcore_axis_name="s")
    kfn = pl.kernel(
        _gather_body,
        out_type=jax.ShapeDtypeStruct((B * M, CP), jnp.float32),
        mesh=mesh,
        scratch_types=[
            pltpu.VMEM((RPC,), jnp.int32),
            pltpu.VMEM((RPC, 16), jnp.float32),
            pltpu.VMEM((RPC, CP), jnp.float32),
            pltpu.VMEM((QC, CP), jnp.float32),
            pltpu.SemaphoreType.DMA,
        ],
    )
    return kfn(table, idx_flat, wexp)


def _regressor_body(agg_ref, qT_ref, gterm_ref, Wr1q_ref, Wr1f_ref,
                    Wr2_ref, br2_ref, Wr3_ref, br3_ref, out_ref):
    q16 = _bf(qT_ref[0])
    a16 = _bf(agg_ref[0])
    pre1 = (jnp.dot(a16, Wr1f_ref[...].T,
                    preferred_element_type=jnp.float32, precision=HI)
            + jnp.dot(q16, Wr1q_ref[...].T,
                      preferred_element_type=jnp.float32, precision=HI)
            + gterm_ref[0])
    h1 = jnp.maximum(pre1, 0.0)
    h2 = (jnp.dot(h1, Wr2_ref[...].T,
                  preferred_element_type=jnp.float32, precision=HI)
          + br2_ref[...][None, :])
    h2 = jnp.maximum(h2, 0.0)
    s = jnp.sum(h2 * Wr3_ref[...], axis=1, keepdims=True)
    out_ref[0] = s + br3_ref[0]


def _regressor(agg, queryT, gterm, Wr1q, Wr1f, Wr2, br2, Wr3, br3):
    rep = lambda x: pl.BlockSpec(x.shape, lambda b: (0,) * x.ndim)
    return pl.pallas_call(
        _regressor_body,
        grid=(B,),
        in_specs=[
            pl.BlockSpec((1, M, CP), lambda b: (b, 0, 0)),
            pl.BlockSpec((1, M, 3), lambda b: (b, 0, 0)),
            pl.BlockSpec((1, 1, 256), lambda b: (b, 0, 0)),
            rep(Wr1q), rep(Wr1f), rep(Wr2), rep(br2), rep(Wr3), rep(br3),
        ],
        out_specs=pl.BlockSpec((1, M, 1), lambda b: (b, 0, 0)),
        out_shape=jax.ShapeDtypeStruct((B, M, 1), jnp.float32),
    )(agg, queryT, gterm, Wr1q, Wr1f, Wr2, br2, Wr3, br3)


def kernel(original_pts, query_pts, W1, b1, W2, b2, W3, b3, Wg, bg,
           Wr1, br1, Wr2, br2, Wr3, br3):
    ptsT = jnp.transpose(original_pts, (0, 2, 1))
    queryT = jnp.transpose(query_pts, (0, 2, 1))
    Wr1q = Wr1[:, :3]
    Wr1f = jnp.concatenate(
        [Wr1[:, 3:3 + CF], jnp.zeros((Wr1.shape[0], CP - CF), Wr1.dtype)],
        axis=1)
    Wr1g = Wr1[:, 3 + CF:]

    fcat, gterm = _features(ptsT, W1, b1, W2, b2, W3, b3, Wg, bg, Wr1g, br1)
    idx, w = _knn(original_pts, ptsT, queryT)
    idx_flat = idx.reshape(B * M * K)
    wexp = jnp.broadcast_to(w.reshape(B * M * K, 1), (B * M * K, 16))
    table = fcat.reshape(B * N, CP)
    agg = _sc_gather(table, idx_flat, wexp)
    out = _regressor(agg.reshape(B, M, CP), queryT, gterm,
                     Wr1q, Wr1f, Wr2, br2, Wr3, br3)
    return jnp.transpose(out, (0, 2, 1))
